# baseline (speedup 1.0000x reference)
import functools

import jax
import jax.numpy as jnp
from jax import lax
from jax.experimental import pallas as pl
from jax.experimental.pallas import tpu as pltpu
from jax.experimental.pallas import tpu_sc as plsc

_CH = 128
_SUBS = 16
_SGE = _CH * _SUBS
_LANES = 16


@functools.lru_cache(maxsize=None)
def _make_sc_kernel(N, E, C):
    info = plsc.get_sparse_core_info()
    NC, NS = info.num_cores, info.num_subcores
    assert E % _SGE == 0 and C % _LANES == 0
    NW = NC * NS
    nsg = E // _SGE
    q, r = divmod(nsg, NW)
    rows_per_tile = ((N + NS - 1) // NS + 7) // 8 * 8
    n_pad = NS * rows_per_tile
    groups = C // _LANES

    mesh = plsc.VectorSubcoreMesh(core_axis_name="c", subcore_axis_name="s")

    @functools.partial(
        pl.kernel,
        out_type=jax.ShapeDtypeStruct((NC, n_pad, C), jnp.float32),
        mesh=mesh,
        scratch_types=[
            pltpu.VMEM((_CH, C), jnp.float32),
            pltpu.VMEM((_CH, C), jnp.float32),
            pltpu.VMEM((3, _SUBS, _CH), jnp.int32),
            pltpu.VMEM_SHARED((n_pad, C), jnp.float32),
            pltpu.SemaphoreType.DMA,
            pltpu.SemaphoreType.DMA,
            pltpu.SemaphoreType.DMA,
            pltpu.SemaphoreType.DMA,
            pltpu.SemaphoreType.DMA,
        ],
        compiler_params=pltpu.CompilerParams(needs_layout_passes=False),
    )
    def sc_kernel(x_hbm, comb_hbm, out_hbm,
                  rows0, rows1, comb_buf, acc,
                  isem, gsem0, gsem1, ssem0, ssem1):
        cid = lax.axis_index("c")
        sid = lax.axis_index("s")
        wid = sid * NC + cid

        zero = jnp.zeros((_LANES,), jnp.float32)

        @plsc.parallel_loop(0, _CH, unroll=8)
        def zero_rows(rr):
            for g in range(groups):
                rows0[rr, pl.ds(g * _LANES, _LANES)] = zero

        row0 = sid * rows_per_tile
        for m in range(0, rows_per_tile, _CH):
            sz = min(_CH, rows_per_tile - m)
            pltpu.sync_copy(rows0.at[pl.ds(0, sz)],
                            acc.at[pl.ds(row0 + m, sz)])
        plsc.subcore_barrier()

        start = wid * q + jnp.minimum(wid, r)
        cnt = q + jnp.where(wid < r, 1, 0)

        rows = (rows0, rows1)
        gsems = (gsem0, gsem1)
        ssems = (ssem0, ssem1)

        def sg_body(sg, _):
            pltpu.async_copy(comb_hbm.at[start + sg], comb_buf, isem).wait()

            gd = [None] * _SUBS
            sd = [None] * _SUBS
            gd[0] = pltpu.async_copy(
                x_hbm.at[comb_buf.at[0, 0]], rows[0], gsems[0])
            for sub in range(_SUBS):
                b = sub & 1
                if sub + 1 < _SUBS:
                    if sub >= 1:
                        sd[sub - 1].wait()
                    gd[sub + 1] = pltpu.async_copy(
                        x_hbm.at[comb_buf.at[0, sub + 1]], rows[1 - b],
                        gsems[1 - b])
                gd[sub].wait()

                @plsc.parallel_loop(0, _CH, unroll=8)
                def mul_body(e, _rows=rows[b], _sub=sub):
                    w16 = plsc.bitcast(
                        plsc.load_gather(
                            comb_buf,
                            [jnp.full((_LANES,), 2, jnp.int32),
                             jnp.full((_LANES,), _sub, jnp.int32),
                             jnp.full((_LANES,), e, jnp.int32)]),
                        jnp.float32)
                    for g in range(groups):
                        sl = pl.ds(g * _LANES, _LANES)
                        _rows[e, sl] = _rows[e, sl] * w16

                sd[sub] = pltpu.async_copy(
                    rows[b], acc.at[comb_buf.at[1, sub]], ssems[b], add=True)
            sd[_SUBS - 2].wait()
            sd[_SUBS - 1].wait()
            return 0

        lax.fori_loop(0, cnt, sg_body, 0)
        plsc.subcore_barrier()

        for m in range(0, rows_per_tile, _CH):
            sz = min(_CH, rows_per_tile - m)
            pltpu.sync_copy(acc.at[pl.ds(row0 + m, sz)],
                            rows0.at[pl.ds(0, sz)])
            pltpu.sync_copy(rows0.at[pl.ds(0, sz)],
                            out_hbm.at[cid, pl.ds(row0 + m, sz)])

    return sc_kernel


def _tc_add(partials, N, C):
    blk = 1000

    def add_body(p_ref, o_ref):
        o_ref[...] = p_ref[0] + p_ref[1]

    return pl.pallas_call(
        add_body,
        out_shape=jax.ShapeDtypeStruct((N, C), jnp.float32),
        grid=(N // blk,),
        in_specs=[pl.BlockSpec((2, blk, C), lambda i: (0, i, 0))],
        out_specs=pl.BlockSpec((blk, C), lambda i: (i, 0)),
    )(partials)


def kernel(x, g, ew):
    N, C = x.shape
    E = ew.shape[0]
    i = g[0].astype(jnp.int32)
    j = g[1].astype(jnp.int32)
    ew = ew.astype(jnp.float32)
    pad = (-E) % _SGE
    if pad:
        i = jnp.concatenate([i, jnp.zeros((pad,), jnp.int32)])
        j = jnp.concatenate([j, jnp.zeros((pad,), jnp.int32)])
        ew = jnp.concatenate([ew, jnp.zeros((pad,), jnp.float32)])
    nsg = (E + pad) // _SGE
    comb = jnp.stack([
        i.reshape(nsg, _SUBS, _CH),
        j.reshape(nsg, _SUBS, _CH),
        lax.bitcast_convert_type(ew, jnp.int32).reshape(nsg, _SUBS, _CH),
    ], axis=1)
    partials = _make_sc_kernel(N, E + pad, C)(x.astype(jnp.float32), comb)
    return _tc_add(partials, N, C)


# speedup vs baseline: 1.1510x; 1.1510x over previous
import functools

import jax
import jax.numpy as jnp
from jax import lax
from jax.experimental import pallas as pl
from jax.experimental.pallas import tpu as pltpu
from jax.experimental.pallas import tpu_sc as plsc

_CH = 128
_SUBS = 8
_SGE = _CH * _SUBS
_LANES = 16


@functools.lru_cache(maxsize=None)
def _make_sc_kernel(N, E, C):
    info = plsc.get_sparse_core_info()
    NC, NS = info.num_cores, info.num_subcores
    assert E % _SGE == 0 and C % _LANES == 0
    NW = NC * NS
    nsg = E // _SGE
    q, r = divmod(nsg, NW)
    rows_per_tile = ((N + NS - 1) // NS + 7) // 8 * 8
    n_pad = NS * rows_per_tile
    groups = C // _LANES

    mesh = plsc.VectorSubcoreMesh(core_axis_name="c", subcore_axis_name="s")

    @functools.partial(
        pl.kernel,
        out_type=jax.ShapeDtypeStruct((NC, n_pad, C), jnp.float32),
        mesh=mesh,
        scratch_types=[
            pltpu.VMEM((_CH, C), jnp.float32),
            pltpu.VMEM((_CH, C), jnp.float32),
            pltpu.VMEM((3, _SUBS, _CH), jnp.int32),
            pltpu.VMEM_SHARED((n_pad, C), jnp.float32),
            pltpu.SemaphoreType.DMA,
            pltpu.SemaphoreType.DMA,
            pltpu.SemaphoreType.DMA,
            pltpu.SemaphoreType.DMA,
            pltpu.SemaphoreType.DMA,
        ],
        compiler_params=pltpu.CompilerParams(needs_layout_passes=False),
    )
    def sc_kernel(x_hbm, comb_hbm, out_hbm,
                  rows0, rows1, comb_buf, acc,
                  isem, gsem0, gsem1, ssem0, ssem1):
        cid = lax.axis_index("c")
        sid = lax.axis_index("s")
        wid = sid * NC + cid

        zero = jnp.zeros((_LANES,), jnp.float32)

        @plsc.parallel_loop(0, _CH, unroll=8)
        def zero_rows(rr):
            for g in range(groups):
                rows0[rr, pl.ds(g * _LANES, _LANES)] = zero

        row0 = sid * rows_per_tile
        for m in range(0, rows_per_tile, _CH):
            sz = min(_CH, rows_per_tile - m)
            pltpu.sync_copy(rows0.at[pl.ds(0, sz)],
                            acc.at[pl.ds(row0 + m, sz)])
        plsc.subcore_barrier()

        start = wid * q + jnp.minimum(wid, r)
        cnt = q + jnp.where(wid < r, 1, 0)

        rows = (rows0, rows1)
        gsems = (gsem0, gsem1)
        ssems = (ssem0, ssem1)

        def sg_body(sg, _):
            pltpu.async_copy(comb_hbm.at[start + sg], comb_buf, isem).wait()

            gd = [None] * _SUBS
            sd = [None] * _SUBS
            gd[0] = pltpu.async_copy(
                x_hbm.at[comb_buf.at[0, 0]], rows[0], gsems[0])
            for sub in range(_SUBS):
                b = sub & 1
                if sub + 1 < _SUBS:
                    if sub >= 1:
                        sd[sub - 1].wait()
                    gd[sub + 1] = pltpu.async_copy(
                        x_hbm.at[comb_buf.at[0, sub + 1]], rows[1 - b],
                        gsems[1 - b])
                gd[sub].wait()

                @plsc.parallel_loop(0, _CH, unroll=8)
                def mul_body(e, _rows=rows[b], _sub=sub):
                    w16 = plsc.bitcast(
                        plsc.load_gather(
                            comb_buf,
                            [jnp.full((_LANES,), 2, jnp.int32),
                             jnp.full((_LANES,), _sub, jnp.int32),
                             jnp.full((_LANES,), e, jnp.int32)]),
                        jnp.float32)
                    for g in range(groups):
                        sl = pl.ds(g * _LANES, _LANES)
                        _rows[e, sl] = _rows[e, sl] * w16

                sd[sub] = pltpu.async_copy(
                    rows[b], acc.at[comb_buf.at[1, sub]], ssems[b], add=True)
            sd[_SUBS - 2].wait()
            sd[_SUBS - 1].wait()
            return 0

        lax.fori_loop(0, cnt, sg_body, 0)
        plsc.subcore_barrier()

        for m in range(0, rows_per_tile, _CH):
            sz = min(_CH, rows_per_tile - m)
            pltpu.sync_copy(acc.at[pl.ds(row0 + m, sz)],
                            rows0.at[pl.ds(0, sz)])
            pltpu.sync_copy(rows0.at[pl.ds(0, sz)],
                            out_hbm.at[cid, pl.ds(row0 + m, sz)])

    return sc_kernel


def _tc_add(partials, N, C):
    blk = 1000

    def add_body(p_ref, o_ref):
        o_ref[...] = p_ref[0] + p_ref[1]

    return pl.pallas_call(
        add_body,
        out_shape=jax.ShapeDtypeStruct((N, C), jnp.float32),
        grid=(N // blk,),
        in_specs=[pl.BlockSpec((2, blk, C), lambda i: (0, i, 0))],
        out_specs=pl.BlockSpec((blk, C), lambda i: (i, 0)),
    )(partials)


def kernel(x, g, ew):
    N, C = x.shape
    E = ew.shape[0]
    i = g[0].astype(jnp.int32)
    j = g[1].astype(jnp.int32)
    ew = ew.astype(jnp.float32)
    pad = (-E) % _SGE
    if pad:
        i = jnp.concatenate([i, jnp.zeros((pad,), jnp.int32)])
        j = jnp.concatenate([j, jnp.zeros((pad,), jnp.int32)])
        ew = jnp.concatenate([ew, jnp.zeros((pad,), jnp.float32)])
    nsg = (E + pad) // _SGE
    comb = jnp.stack([
        i.reshape(nsg, _SUBS, _CH),
        j.reshape(nsg, _SUBS, _CH),
        lax.bitcast_convert_type(ew, jnp.int32).reshape(nsg, _SUBS, _CH),
    ], axis=1)
    partials = _make_sc_kernel(N, E + pad, C)(x.astype(jnp.float32), comb)
    return _tc_add(partials, N, C)
